# speedup vs baseline: 1.8087x; 1.8087x over previous
import functools

import jax
import jax.numpy as jnp
from jax.experimental import pallas as pl
from jax.experimental.pallas import tpu as pltpu

B, J, V = 32, 8, 32768
VC = 2048
NCHUNK = V // VC


def _fused_body(a_ref, e_ref, logits_ref, stddev_ref, best_ref, m_ref, i_ref):
    j = pl.program_id(0)

    a = a_ref[...]
    e = e_ref[...]

    m = a[:, 0, :]
    s = e[:, 0, :]
    for jj in range(1, J):
        aj = a[:, jj, :]
        upd = aj < m
        m = jnp.where(upd, aj, m)
        s = jnp.where(upd, e[:, jj, :], s)
    logits_ref[...] = m
    stddev_ref[...] = s

    @pl.when(j == 0)
    def _():
        m_ref[...] = jnp.full((B, J), -jnp.inf, jnp.float32)
        i_ref[...] = jnp.zeros((B, J), jnp.int32)

    cm = jnp.max(a, axis=2)
    iota = jax.lax.broadcasted_iota(jnp.int32, (B, J, VC), 2) + j * VC
    li = jnp.min(jnp.where(a == cm[:, :, None], iota, V), axis=2)
    upd = cm > m_ref[...]
    m_ref[...] = jnp.where(upd, cm, m_ref[...])
    i_ref[...] = jnp.where(upd, li, i_ref[...])
    best_ref[...] = i_ref[...]


@jax.jit
def kernel(action, explore_rate):
    logits, stddev, best2d = pl.pallas_call(
        _fused_body,
        grid=(NCHUNK,),
        in_specs=[
            pl.BlockSpec((B, J, VC), lambda j: (0, 0, j)),
            pl.BlockSpec((B, J, VC), lambda j: (0, 0, j)),
        ],
        out_specs=[
            pl.BlockSpec((B, VC), lambda j: (0, j)),
            pl.BlockSpec((B, VC), lambda j: (0, j)),
            pl.BlockSpec((B, J), lambda j: (0, 0)),
        ],
        out_shape=[
            jax.ShapeDtypeStruct((B, V), jnp.float32),
            jax.ShapeDtypeStruct((B, V), jnp.float32),
            jax.ShapeDtypeStruct((B, J), jnp.int32),
        ],
        scratch_shapes=[
            pltpu.VMEM((B, J), jnp.float32),
            pltpu.VMEM((B, J), jnp.int32),
        ],
        compiler_params=pltpu.CompilerParams(
            dimension_semantics=("arbitrary",),
        ),
    )(action, explore_rate)
    return logits, stddev, best2d[:, None, :]
